# baseline (speedup 1.0000x reference)
import functools

import jax
import jax.numpy as jnp
from jax import lax
from jax.experimental import pallas as pl
from jax.experimental.pallas import tpu as pltpu
from jax.experimental.pallas import tpu_sc as plsc

N = 10000
E = 320000
D = 128
A = 16
CDIM = 16

NW = 32
K = 128
EW = 10240
KB = EW // K
EP = NW * EW
NP = 10240
ROWS_PER_TILE = NP // 16

BE = 2048
BN = 1024

_MESH = plsc.VectorSubcoreMesh(core_axis_name="c", subcore_axis_name="s")


def _silu(x):
    return x * jax.nn.sigmoid(x)


DP = D // 2


def _pack_bf16(x):
    lo = jax.lax.bitcast_convert_type(
        x[:, :DP].astype(jnp.bfloat16).astype(jnp.float32), jnp.uint32)
    hi = jax.lax.bitcast_convert_type(
        x[:, DP:].astype(jnp.bfloat16).astype(jnp.float32), jnp.uint32)
    return jax.lax.bitcast_convert_type((lo >> 16) | hi, jnp.int32)


def _unpack_bf16(p):
    u = jax.lax.bitcast_convert_type(p, jnp.uint32)
    lo = jax.lax.bitcast_convert_type(u << 16, jnp.float32)
    hi = jax.lax.bitcast_convert_type(u & jnp.uint32(0xFFFF0000),
                                      jnp.float32)
    return jnp.concatenate([lo, hi], axis=1)


GSETS = 4


def _gather_body(ha, hb, co, idxr3, idxc3, gha, ghb, gcd,
                 idx_r, idx_c,
                 a0, b0, r0, c0, o0,
                 a1, b1, r1, c1, o1,
                 a2, b2, r2, c2, o2,
                 a3, b3, r3, c3, o3,
                 sg0, sg1, sg2, sg3, sw0, sw1, sw2, sw3):
    c = lax.axis_index("c")
    s = lax.axis_index("s")
    wid = s * 2 + c
    base = wid * EW
    gA = (a0, a1, a2, a3)
    gB = (b0, b1, b2, b3)
    gR = (r0, r1, r2, r3)
    gC = (c0, c1, c2, c3)
    oC = (o0, o1, o2, o3)
    sem_g = (sg0, sg1, sg2, sg3)
    sem_w = (sw0, sw1, sw2, sw3)
    pltpu.sync_copy(idxr3.at[wid], idx_r)
    pltpu.sync_copy(idxc3.at[wid], idx_c)

    def fire_g(sl, b):
        pltpu.async_copy(ha.at[idx_r.at[sl]], gA[b], sem_g[b])
        pltpu.async_copy(hb.at[idx_c.at[sl]], gB[b], sem_g[b])
        pltpu.async_copy(co.at[idx_r.at[sl]], gR[b], sem_g[b])
        pltpu.async_copy(co.at[idx_c.at[sl]], gC[b], sem_g[b])

    def wait_g(sl, b):
        pltpu.make_async_copy(ha.at[idx_r.at[sl]], gA[b], sem_g[b]).wait()
        pltpu.make_async_copy(hb.at[idx_c.at[sl]], gB[b], sem_g[b]).wait()
        pltpu.make_async_copy(co.at[idx_r.at[sl]], gR[b], sem_g[b]).wait()
        pltpu.make_async_copy(co.at[idx_c.at[sl]], gC[b], sem_g[b]).wait()

    def fire_w(t, b):
        off = base + t * K
        pltpu.async_copy(gA[b], gha.at[pl.ds(off, K)], sem_w[b])
        pltpu.async_copy(gB[b], ghb.at[pl.ds(off, K)], sem_w[b])
        pltpu.async_copy(oC[b], gcd.at[pl.ds(off, K)], sem_w[b])

    def wait_w(t, b):
        off = base + t * K
        pltpu.make_async_copy(gA[b], gha.at[pl.ds(off, K)], sem_w[b]).wait()
        pltpu.make_async_copy(gB[b], ghb.at[pl.ds(off, K)], sem_w[b]).wait()
        pltpu.make_async_copy(oC[b], gcd.at[pl.ds(off, K)], sem_w[b]).wait()

    def compute_cd(b):
        cr, cc, oc = gR[b], gC[b], oC[b]

        def row(r, carry):
            oc[r, :] = cr[r, :] - cc[r, :]
            return carry

        lax.fori_loop(0, K, row, 0)

    fire_g(0, 0)

    def step(t, b, do_wait_w, do_fire_g):
        bn = (b + 1) % GSETS
        if do_wait_w:
            wait_w(t + 1 - GSETS, bn)
        if do_fire_g:
            fire_g(t + 1, bn)
        wait_g(t, b)
        compute_cd(b)
        fire_w(t, b)

    for t in range(GSETS):
        step(t, t, t == GSETS - 1, True)

    def loop(i, carry):
        for b in range(GSETS):
            step(GSETS * i + b, b, True, True)
        return carry

    lax.fori_loop(1, KB // GSETS - 1, loop, 0)
    for b in range(GSETS):
        t = KB - GSETS + b
        step(t, b, True, b < GSETS - 1)
    for b in range(1, GSETS):
        wait_w(KB - GSETS + b, b)


def _sc_gather(ha, hb, co, idxr3, idxc3):
    f32 = jnp.float32
    i32 = jnp.int32
    out_type = (
        jax.ShapeDtypeStruct((EP, DP), i32),
        jax.ShapeDtypeStruct((EP, DP), i32),
        jax.ShapeDtypeStruct((EP, CDIM), f32),
    )
    setbufs = [
        pltpu.VMEM((K, DP), i32),
        pltpu.VMEM((K, DP), i32),
        pltpu.VMEM((K, CDIM), f32),
        pltpu.VMEM((K, CDIM), f32),
        pltpu.VMEM((K, CDIM), f32),
    ]
    scratch = ([pltpu.VMEM((KB, K), jnp.int32),
                pltpu.VMEM((KB, K), jnp.int32)]
               + setbufs * GSETS
               + [pltpu.SemaphoreType.DMA] * (2 * GSETS))
    fn = pl.kernel(_gather_body, out_type=out_type, mesh=_MESH,
                   scratch_types=scratch,
                   compiler_params=pltpu.CompilerParams(
                       use_tc_tiling_on_sc=False))
    return fn(ha, hb, co, idxr3, idxc3)


SSETS = 2
KS = 64
SSTEPS = EW // KS


def _scatter_body(m, wcd, idx3, hpart, cpart,
                  idx_r, m0, c0, m1, c1, acc_h, acc_c,
                  si0, si1, ss0, ss1):
    c = lax.axis_index("c")
    s = lax.axis_index("s")
    wid = s * 2 + c
    base = wid * EW
    pltpu.sync_copy(idx3.at[wid], idx_r)
    mb = (m0, m1)
    cb = (c0, c1)
    sem_i = (si0, si1)
    sem_s = (ss0, ss1)

    zero16 = jnp.zeros((16,), jnp.float32)

    def zrow(i, carry):
        for j in range(D // 16):
            m0[i, pl.ds(j * 16, 16)] = zero16
        c0[i, :] = zero16
        return carry

    lax.fori_loop(0, KS, zrow, 0)
    tile_row0 = s * ROWS_PER_TILE
    for k in range(ROWS_PER_TILE // KS):
        pltpu.sync_copy(m0, acc_h.at[pl.ds(tile_row0 + k * KS, KS)])
        pltpu.sync_copy(c0, acc_c.at[pl.ds(tile_row0 + k * KS, KS)])
    plsc.subcore_barrier()

    def fire_in(t, b):
        off = base + t * KS
        pltpu.async_copy(m.at[pl.ds(off, KS)], mb[b], sem_i[b])
        pltpu.async_copy(wcd.at[pl.ds(off, KS)], cb[b], sem_i[b])

    def wait_in(t, b):
        off = base + t * KS
        pltpu.make_async_copy(m.at[pl.ds(off, KS)], mb[b], sem_i[b]).wait()
        pltpu.make_async_copy(wcd.at[pl.ds(off, KS)], cb[b],
                              sem_i[b]).wait()

    def fire_sc(sl, b):
        pltpu.async_copy(mb[b], acc_h.at[idx_r.at[sl]], sem_s[b], add=True)
        pltpu.async_copy(cb[b], acc_c.at[idx_r.at[sl]], sem_s[b], add=True)

    def wait_sc(sl, b):
        pltpu.make_async_copy(mb[b], acc_h.at[idx_r.at[sl]],
                              sem_s[b]).wait()
        pltpu.make_async_copy(cb[b], acc_c.at[idx_r.at[sl]],
                              sem_s[b]).wait()

    fire_in(0, 0)

    def step(t, b, do_wait_sc, do_fire_in):
        bn = (b + 1) % SSETS
        if do_wait_sc:
            wait_sc(t + 1 - SSETS, bn)
        if do_fire_in:
            fire_in(t + 1, bn)
        wait_in(t, b)
        fire_sc(t, b)

    for t in range(SSETS):
        step(t, t, t == SSETS - 1, True)

    def loop(i, carry):
        for b in range(SSETS):
            step(SSETS * i + b, b, True, True)
        return carry

    lax.fori_loop(1, SSTEPS // SSETS - 1, loop, 0)
    for b in range(SSETS):
        t = SSTEPS - SSETS + b
        step(t, b, True, b < SSETS - 1)
    for b in range(1, SSETS):
        wait_sc(SSTEPS - SSETS + b, b)

    plsc.subcore_barrier()
    pltpu.sync_copy(acc_h.at[pl.ds(tile_row0, ROWS_PER_TILE)],
                    hpart.at[c, pl.ds(tile_row0, ROWS_PER_TILE)])
    pltpu.sync_copy(acc_c.at[pl.ds(tile_row0, ROWS_PER_TILE)],
                    cpart.at[c, pl.ds(tile_row0, ROWS_PER_TILE)])


def _sc_scatter(m, wcd, idx3):
    f32 = jnp.float32
    out_type = (jax.ShapeDtypeStruct((2, NP, D), f32),
                jax.ShapeDtypeStruct((2, NP, CDIM), f32))
    scratch = ([pltpu.VMEM((SSTEPS, KS), jnp.int32)]
               + [pltpu.VMEM((KS, D), f32),
                  pltpu.VMEM((KS, CDIM), f32)] * SSETS
               + [pltpu.VMEM_SHARED((NP, D), f32),
                  pltpu.VMEM_SHARED((NP, CDIM), f32)]
               + [pltpu.SemaphoreType.DMA] * (2 * SSETS))
    fn = pl.kernel(_scatter_body, out_type=out_type, mesh=_MESH,
                   scratch_types=scratch,
                   compiler_params=pltpu.CompilerParams(
                       use_tc_tiling_on_sc=False))
    return fn(m, wcd, idx3)


def _full(shape):
    return pl.BlockSpec(shape, lambda i: tuple(0 for _ in shape))


def _edge_tc(gha, ghb, gcd, ea, We, W1, Wc0, bias):
    def body(gha_r, ghb_r, gcd_r, ea_r, We_r, W1_r, Wc0_r, b_r,
             m_r, wcd_r):
        cd = gcd_r[...]
        radial = jnp.sum(cd * cd, axis=1, keepdims=True)
        b0 = b_r[0:1, :]
        b1 = b_r[1:2, :]
        bc0 = b_r[2:3, :]
        wc1 = b_r[3:4, :]
        wr = b_r[4:5, :]
        gsum = _unpack_bf16(gha_r[...]) + _unpack_bf16(ghb_r[...])
        mpre = (gsum + radial * wr + b0
                + jnp.dot(ea_r[...], We_r[...],
                          preferred_element_type=jnp.float32))
        bf = jnp.bfloat16
        m0 = _silu(mpre)
        m = _silu(jnp.dot(m0.astype(bf), W1_r[...].astype(bf),
                          preferred_element_type=jnp.float32) + b1)
        th = _silu(jnp.dot(m.astype(bf), Wc0_r[...].astype(bf),
                           preferred_element_type=jnp.float32) + bc0)
        t = jnp.sum(th * wc1, axis=1, keepdims=True)
        m_r[...] = m
        lane = lax.broadcasted_iota(jnp.int32, (BE, CDIM), 1)
        wcd_r[...] = jnp.where(lane == 3, 1.0, cd * t)

    grid = (EP // BE,)
    return pl.pallas_call(
        body,
        grid=grid,
        in_specs=[
            pl.BlockSpec((BE, DP), lambda i: (i, 0)),
            pl.BlockSpec((BE, DP), lambda i: (i, 0)),
            pl.BlockSpec((BE, CDIM), lambda i: (i, 0)),
            pl.BlockSpec((BE, A), lambda i: (i, 0)),
            _full((A, D)),
            _full((D, D)),
            _full((D, D)),
            _full((8, D)),
        ],
        out_specs=[
            pl.BlockSpec((BE, D), lambda i: (i, 0)),
            pl.BlockSpec((BE, CDIM), lambda i: (i, 0)),
        ],
        out_shape=[
            jax.ShapeDtypeStruct((EP, D), jnp.float32),
            jax.ShapeDtypeStruct((EP, CDIM), jnp.float32),
        ],
    )(gha, ghb, gcd, ea, We, W1, Wc0, bias)


def _node_tc(h, hp0, hp1, co, cp0, cp1, Wn0a, Wn0b, Wn1, Wax, Wbx, bias,
             last):
    def body(h_r, hp0_r, hp1_r, co_r, cp0_r, cp1_r,
             Wn0a_r, Wn0b_r, Wn1_r, Wax_r, Wbx_r, b_r, *outs):
        h = h_r[...]
        agg = hp0_r[...] + hp1_r[...]
        bn0 = b_r[0:1, :]
        bn1 = b_r[1:2, :]
        o = _silu(jnp.dot(h, Wn0a_r[...], preferred_element_type=jnp.float32)
                  + jnp.dot(agg, Wn0b_r[...],
                            preferred_element_type=jnp.float32) + bn0)
        o = jnp.dot(o, Wn1_r[...], preferred_element_type=jnp.float32) + bn1
        hn = h + o
        csum = cp0_r[...] + cp1_r[...]
        cnt = jnp.clip(csum[:, 3:4], 1.0, None)
        upd = csum / cnt
        lane = lax.broadcasted_iota(jnp.int32, (BN, CDIM), 1)
        co_new = co_r[...] + jnp.where(lane < 3, upd, 0.0)
        if last:
            hf_r, co_r_out = outs
            hf_r[...] = (jnp.dot(hn, Wax_r[...],
                                 preferred_element_type=jnp.float32)
                         + b_r[2:3, :])
            co_r_out[...] = co_new
        else:
            hn_r, co_r_out, ha_r, hb_r = outs
            hn_r[...] = hn
            co_r_out[...] = co_new
            ha_r[...] = _pack_bf16(jnp.dot(hn, Wax_r[...],
                                           preferred_element_type=jnp.float32))
            hb_r[...] = _pack_bf16(jnp.dot(hn, Wbx_r[...],
                                           preferred_element_type=jnp.float32))

    grid = (NP // BN,)
    nd = pl.BlockSpec((BN, D), lambda i: (i, 0))
    ndc = pl.BlockSpec((BN, CDIM), lambda i: (i, 0))
    if last:
        out_specs = [nd, ndc]
        out_shape = [jax.ShapeDtypeStruct((NP, D), jnp.float32),
                     jax.ShapeDtypeStruct((NP, CDIM), jnp.float32)]
    else:
        ndp = pl.BlockSpec((BN, DP), lambda i: (i, 0))
        out_specs = [nd, ndc, ndp, ndp]
        out_shape = [jax.ShapeDtypeStruct((NP, D), jnp.float32),
                     jax.ShapeDtypeStruct((NP, CDIM), jnp.float32),
                     jax.ShapeDtypeStruct((NP, DP), jnp.int32),
                     jax.ShapeDtypeStruct((NP, DP), jnp.int32)]
    return pl.pallas_call(
        body,
        grid=grid,
        in_specs=[nd, nd, nd, ndc, ndc, ndc,
                  _full((D, D)), _full((D, D)), _full((D, D)),
                  _full((D, D)), _full((D, D)), _full((8, D))],
        out_specs=out_specs,
        out_shape=out_shape,
    )(h, hp0, hp1, co, cp0, cp1, Wn0a, Wn0b, Wn1, Wax, Wbx, bias)


def _init_tc(xp, Wemb, Wa0, Wb0, bias):
    def body(x_r, Wemb_r, Wa_r, Wb_r, b_r, h_r, ha_r, hb_r):
        h = (jnp.dot(x_r[...], Wemb_r[...],
                     preferred_element_type=jnp.float32) + b_r[0:1, :])
        h_r[...] = h
        ha_r[...] = _pack_bf16(jnp.dot(h, Wa_r[...],
                                       preferred_element_type=jnp.float32))
        hb_r[...] = _pack_bf16(jnp.dot(h, Wb_r[...],
                                       preferred_element_type=jnp.float32))

    grid = (NP // BN,)
    nd = pl.BlockSpec((BN, D), lambda i: (i, 0))
    ndp = pl.BlockSpec((BN, DP), lambda i: (i, 0))
    return pl.pallas_call(
        body,
        grid=grid,
        in_specs=[nd, _full((D, D)), _full((D, D)), _full((D, D)),
                  _full((8, D))],
        out_specs=[nd, ndp, ndp],
        out_shape=[jax.ShapeDtypeStruct((NP, D), jnp.float32),
                   jax.ShapeDtypeStruct((NP, DP), jnp.int32),
                   jax.ShapeDtypeStruct((NP, DP), jnp.int32)],
    )(xp, Wemb, Wa0, Wb0, bias)


def _bias_stack(rows):
    stack = jnp.stack(rows, axis=0)
    return jnp.pad(stack, ((0, 8 - stack.shape[0]), (0, 0)))


def kernel(x, pos, edge_index, edge_attr, params):
    f32 = jnp.float32
    xp = jnp.pad(x.astype(f32), ((0, NP - N), (0, 0)))
    co = jnp.zeros((NP, CDIM), f32).at[:N, :3].set(pos.astype(f32))
    row = edge_index[0].astype(jnp.int32)
    col = edge_index[1].astype(jnp.int32)
    rowp = jnp.pad(row, (0, EP - E), constant_values=N)
    idxr3 = rowp.reshape(NW, KB, K)
    idxr2 = rowp.reshape(NW, EW // 64, 64)
    idxc3 = jnp.pad(col, (0, EP - E), constant_values=N).reshape(NW, KB, K)
    eap = jnp.pad(edge_attr.astype(f32), ((0, EP - E), (0, 0)))

    layers = params["layers"]
    l0 = layers[0]
    W0 = l0["edge_mlp0"]["W"]
    init_bias = _bias_stack([params["emb_in"]["b"]])
    h, ha, hb = _init_tc(xp, params["emb_in"]["W"],
                         W0[:D, :], W0[D:2 * D, :], init_bias)

    n_layers = len(layers)
    for li, lp in enumerate(layers):
        W0 = lp["edge_mlp0"]["W"]
        edge_bias = _bias_stack([
            lp["edge_mlp0"]["b"],
            lp["edge_mlp1"]["b"],
            lp["coord_mlp0"]["b"],
            lp["coord_mlp1"]["W"][:, 0],
            W0[2 * D, :],
        ])
        gha, ghb, gcd = _sc_gather(ha, hb, co, idxr3, idxc3)
        m, wcd = _edge_tc(gha, ghb, gcd, eap,
                          W0[2 * D + 1:, :], lp["edge_mlp1"]["W"],
                          lp["coord_mlp0"]["W"], edge_bias)
        hpart, cpart = _sc_scatter(m, wcd, idxr2)
        Wn = lp["node_mlp0"]["W"]
        last = li == n_layers - 1
        if last:
            node_bias = _bias_stack([lp["node_mlp0"]["b"],
                                     lp["node_mlp1"]["b"],
                                     params["emb_out"]["b"]])
            hf, co = _node_tc(h, hpart[0], hpart[1], co, cpart[0], cpart[1],
                              Wn[:D, :], Wn[D:, :], lp["node_mlp1"]["W"],
                              params["emb_out"]["W"], params["emb_out"]["W"],
                              node_bias, True)
        else:
            node_bias = _bias_stack([lp["node_mlp0"]["b"],
                                     lp["node_mlp1"]["b"]])
            Wnext = layers[li + 1]["edge_mlp0"]["W"]
            h, co, ha, hb = _node_tc(h, hpart[0], hpart[1], co,
                                     cpart[0], cpart[1],
                                     Wn[:D, :], Wn[D:, :],
                                     lp["node_mlp1"]["W"],
                                     Wnext[:D, :], Wnext[D:2 * D, :],
                                     node_bias, False)

    return co[:N, :3], hf[:N, :]


# speedup vs baseline: 1.2422x; 1.2422x over previous
import functools

import jax
import jax.numpy as jnp
from jax import lax
from jax.experimental import pallas as pl
from jax.experimental.pallas import tpu as pltpu
from jax.experimental.pallas import tpu_sc as plsc

N = 10000
E = 320000
D = 128
A = 16
CDIM = 16

NW = 32
K = 128
EW = 10240
KB = EW // K
EP = NW * EW
NP = 10240
ROWS_PER_TILE = NP // 16

BE = 2048
BN = 1024

_MESH = plsc.VectorSubcoreMesh(core_axis_name="c", subcore_axis_name="s")


def _silu(x):
    return x * jax.nn.sigmoid(x)


DP = D // 2


def _pack_bf16(x):
    lo = jax.lax.bitcast_convert_type(
        x[:, :DP].astype(jnp.bfloat16).astype(jnp.float32), jnp.uint32)
    hi = jax.lax.bitcast_convert_type(
        x[:, DP:].astype(jnp.bfloat16).astype(jnp.float32), jnp.uint32)
    return jax.lax.bitcast_convert_type((lo >> 16) | hi, jnp.int32)


def _unpack_bf16(p):
    u = jax.lax.bitcast_convert_type(p, jnp.uint32)
    lo = jax.lax.bitcast_convert_type(u << 16, jnp.float32)
    hi = jax.lax.bitcast_convert_type(u & jnp.uint32(0xFFFF0000),
                                      jnp.float32)
    return jnp.concatenate([lo, hi], axis=1)


GSETS = 4


def _gather_body(ha, hb, co, idxr3, idxc3, gha, gcd,
                 idx_r, idx_c,
                 a0, b0, r0, c0, o0,
                 a1, b1, r1, c1, o1,
                 a2, b2, r2, c2, o2,
                 a3, b3, r3, c3, o3,
                 sg0, sg1, sg2, sg3, sw0, sw1, sw2, sw3):
    c = lax.axis_index("c")
    s = lax.axis_index("s")
    wid = s * 2 + c
    base = wid * EW
    gA = (a0, a1, a2, a3)
    gB = (b0, b1, b2, b3)
    gR = (r0, r1, r2, r3)
    gC = (c0, c1, c2, c3)
    oC = (o0, o1, o2, o3)
    sem_g = (sg0, sg1, sg2, sg3)
    sem_w = (sw0, sw1, sw2, sw3)
    pltpu.sync_copy(idxr3.at[wid], idx_r)
    pltpu.sync_copy(idxc3.at[wid], idx_c)

    def fire_g(sl, b):
        pltpu.async_copy(ha.at[idx_r.at[sl]], gA[b], sem_g[b])
        pltpu.async_copy(hb.at[idx_c.at[sl]], gB[b], sem_g[b])
        pltpu.async_copy(co.at[idx_r.at[sl]], gR[b], sem_g[b])
        pltpu.async_copy(co.at[idx_c.at[sl]], gC[b], sem_g[b])

    def wait_g(sl, b):
        pltpu.make_async_copy(ha.at[idx_r.at[sl]], gA[b], sem_g[b]).wait()
        pltpu.make_async_copy(hb.at[idx_c.at[sl]], gB[b], sem_g[b]).wait()
        pltpu.make_async_copy(co.at[idx_r.at[sl]], gR[b], sem_g[b]).wait()
        pltpu.make_async_copy(co.at[idx_c.at[sl]], gC[b], sem_g[b]).wait()

    def fire_w(t, b):
        off = base + t * K
        pltpu.async_copy(gA[b], gha.at[pl.ds(off, K), pl.ds(0, DP)],
                         sem_w[b])
        pltpu.async_copy(gB[b], gha.at[pl.ds(off, K), pl.ds(DP, DP)],
                         sem_w[b])
        pltpu.async_copy(oC[b], gcd.at[pl.ds(off, K)], sem_w[b])

    def wait_w(t, b):
        off = base + t * K
        pltpu.make_async_copy(gA[b], gha.at[pl.ds(off, K), pl.ds(0, DP)],
                              sem_w[b]).wait()
        pltpu.make_async_copy(gB[b], gha.at[pl.ds(off, K), pl.ds(DP, DP)],
                              sem_w[b]).wait()
        pltpu.make_async_copy(oC[b], gcd.at[pl.ds(off, K)], sem_w[b]).wait()

    def compute_cd(b):
        cr, cc, oc = gR[b], gC[b], oC[b]

        def row(r, carry):
            oc[r, :] = cr[r, :] - cc[r, :]
            return carry

        lax.fori_loop(0, K, row, 0)

    fire_g(0, 0)

    def step(t, b, do_wait_w, do_fire_g):
        bn = (b + 1) % GSETS
        if do_wait_w:
            wait_w(t + 1 - GSETS, bn)
        if do_fire_g:
            fire_g(t + 1, bn)
        wait_g(t, b)
        compute_cd(b)
        fire_w(t, b)

    for t in range(GSETS):
        step(t, t, t == GSETS - 1, True)

    def loop(i, carry):
        for b in range(GSETS):
            step(GSETS * i + b, b, True, True)
        return carry

    lax.fori_loop(1, KB // GSETS - 1, loop, 0)
    for b in range(GSETS):
        t = KB - GSETS + b
        step(t, b, True, b < GSETS - 1)
    for b in range(1, GSETS):
        wait_w(KB - GSETS + b, b)


def _sc_gather(ha, hb, co, idxr3, idxc3):
    f32 = jnp.float32
    i32 = jnp.int32
    out_type = (
        jax.ShapeDtypeStruct((EP, D), i32),
        jax.ShapeDtypeStruct((EP, CDIM), f32),
    )
    setbufs = [
        pltpu.VMEM((K, DP), i32),
        pltpu.VMEM((K, DP), i32),
        pltpu.VMEM((K, CDIM), f32),
        pltpu.VMEM((K, CDIM), f32),
        pltpu.VMEM((K, CDIM), f32),
    ]
    scratch = ([pltpu.VMEM((KB, K), jnp.int32),
                pltpu.VMEM((KB, K), jnp.int32)]
               + setbufs * GSETS
               + [pltpu.SemaphoreType.DMA] * (2 * GSETS))
    fn = pl.kernel(_gather_body, out_type=out_type, mesh=_MESH,
                   scratch_types=scratch,
                   compiler_params=pltpu.CompilerParams(
                       use_tc_tiling_on_sc=False))
    return fn(ha, hb, co, idxr3, idxc3)


SSETS = 2
KS = 64
SSTEPS = EW // KS


def _scatter_body(m, wcd, idx3, hpart, cpart,
                  idx_r, m0, c0, m1, c1, acc_h, acc_c,
                  si0, si1, ss0, ss1):
    c = lax.axis_index("c")
    s = lax.axis_index("s")
    wid = s * 2 + c
    base = wid * EW
    pltpu.sync_copy(idx3.at[wid], idx_r)
    mb = (m0, m1)
    cb = (c0, c1)
    sem_i = (si0, si1)
    sem_s = (ss0, ss1)

    zero16 = jnp.zeros((16,), jnp.float32)

    def zrow(i, carry):
        for j in range(D // 16):
            m0[i, pl.ds(j * 16, 16)] = zero16
        c0[i, :] = zero16
        return carry

    lax.fori_loop(0, KS, zrow, 0)
    tile_row0 = s * ROWS_PER_TILE
    for k in range(ROWS_PER_TILE // KS):
        pltpu.sync_copy(m0, acc_h.at[pl.ds(tile_row0 + k * KS, KS)])
        pltpu.sync_copy(c0, acc_c.at[pl.ds(tile_row0 + k * KS, KS)])
    plsc.subcore_barrier()

    def fire_in(t, b):
        off = base + t * KS
        pltpu.async_copy(m.at[pl.ds(off, KS)], mb[b], sem_i[b])
        pltpu.async_copy(wcd.at[pl.ds(off, KS)], cb[b], sem_i[b])

    def wait_in(t, b):
        off = base + t * KS
        pltpu.make_async_copy(m.at[pl.ds(off, KS)], mb[b], sem_i[b]).wait()
        pltpu.make_async_copy(wcd.at[pl.ds(off, KS)], cb[b],
                              sem_i[b]).wait()

    def fire_sc(sl, b):
        pltpu.async_copy(mb[b], acc_h.at[idx_r.at[sl]], sem_s[b], add=True)
        pltpu.async_copy(cb[b], acc_c.at[idx_r.at[sl]], sem_s[b], add=True)

    def wait_sc(sl, b):
        pltpu.make_async_copy(mb[b], acc_h.at[idx_r.at[sl]],
                              sem_s[b]).wait()
        pltpu.make_async_copy(cb[b], acc_c.at[idx_r.at[sl]],
                              sem_s[b]).wait()

    fire_in(0, 0)

    def step(t, b, do_wait_sc, do_fire_in):
        bn = (b + 1) % SSETS
        if do_wait_sc:
            wait_sc(t + 1 - SSETS, bn)
        if do_fire_in:
            fire_in(t + 1, bn)
        wait_in(t, b)
        fire_sc(t, b)

    for t in range(SSETS):
        step(t, t, t == SSETS - 1, True)

    def loop(i, carry):
        for b in range(SSETS):
            step(SSETS * i + b, b, True, True)
        return carry

    lax.fori_loop(1, SSTEPS // SSETS - 1, loop, 0)
    for b in range(SSETS):
        t = SSTEPS - SSETS + b
        step(t, b, True, b < SSETS - 1)
    for b in range(1, SSETS):
        wait_sc(SSTEPS - SSETS + b, b)

    plsc.subcore_barrier()
    pltpu.sync_copy(acc_h.at[pl.ds(tile_row0, ROWS_PER_TILE)],
                    hpart.at[c, pl.ds(tile_row0, ROWS_PER_TILE)])
    pltpu.sync_copy(acc_c.at[pl.ds(tile_row0, ROWS_PER_TILE)],
                    cpart.at[c, pl.ds(tile_row0, ROWS_PER_TILE)])


def _sc_scatter(m, wcd, idx3):
    f32 = jnp.float32
    out_type = (jax.ShapeDtypeStruct((2, NP, D), f32),
                jax.ShapeDtypeStruct((2, NP, CDIM), f32))
    scratch = ([pltpu.VMEM((SSTEPS, KS), jnp.int32)]
               + [pltpu.VMEM((KS, D), f32),
                  pltpu.VMEM((KS, CDIM), f32)] * SSETS
               + [pltpu.VMEM_SHARED((NP, D), f32),
                  pltpu.VMEM_SHARED((NP, CDIM), f32)]
               + [pltpu.SemaphoreType.DMA] * (2 * SSETS))
    fn = pl.kernel(_scatter_body, out_type=out_type, mesh=_MESH,
                   scratch_types=scratch,
                   compiler_params=pltpu.CompilerParams(
                       use_tc_tiling_on_sc=False))
    return fn(m, wcd, idx3)


def _full(shape):
    return pl.BlockSpec(shape, lambda i: tuple(0 for _ in shape))


def _edge_tc(ghab, gcd, ea, We, W1, Wc0, bias):
    def body(ghab_r, gcd_r, ea_r, We_r, W1_r, Wc0_r, b_r,
             m_r, wcd_r):
        cd = gcd_r[...]
        radial = jnp.sum(cd * cd, axis=1, keepdims=True)
        b0 = b_r[0:1, :]
        b1 = b_r[1:2, :]
        bc0 = b_r[2:3, :]
        wc1 = b_r[3:4, :]
        wr = b_r[4:5, :]
        blk = ghab_r[...]
        gsum = _unpack_bf16(blk[:, :DP]) + _unpack_bf16(blk[:, DP:])
        mpre = (gsum + radial * wr + b0
                + jnp.dot(ea_r[...], We_r[...],
                          preferred_element_type=jnp.float32))
        bf = jnp.bfloat16
        m0 = _silu(mpre)
        m = _silu(jnp.dot(m0.astype(bf), W1_r[...].astype(bf),
                          preferred_element_type=jnp.float32) + b1)
        th = _silu(jnp.dot(m.astype(bf), Wc0_r[...].astype(bf),
                           preferred_element_type=jnp.float32) + bc0)
        t = jnp.sum(th * wc1, axis=1, keepdims=True)
        m_r[...] = m
        lane = lax.broadcasted_iota(jnp.int32, (BE, CDIM), 1)
        wcd_r[...] = jnp.where(lane == 3, 1.0, cd * t)

    grid = (EP // BE,)
    return pl.pallas_call(
        body,
        grid=grid,
        in_specs=[
            pl.BlockSpec((BE, D), lambda i: (i, 0)),
            pl.BlockSpec((BE, CDIM), lambda i: (i, 0)),
            pl.BlockSpec((BE, A), lambda i: (i, 0)),
            _full((A, D)),
            _full((D, D)),
            _full((D, D)),
            _full((8, D)),
        ],
        out_specs=[
            pl.BlockSpec((BE, D), lambda i: (i, 0)),
            pl.BlockSpec((BE, CDIM), lambda i: (i, 0)),
        ],
        out_shape=[
            jax.ShapeDtypeStruct((EP, D), jnp.float32),
            jax.ShapeDtypeStruct((EP, CDIM), jnp.float32),
        ],
    )(ghab, gcd, ea, We, W1, Wc0, bias)


def _node_tc(h, hp0, hp1, co, cp0, cp1, Wn0a, Wn0b, Wn1, Wax, Wbx, bias,
             last):
    def body(h_r, hp0_r, hp1_r, co_r, cp0_r, cp1_r,
             Wn0a_r, Wn0b_r, Wn1_r, Wax_r, Wbx_r, b_r, *outs):
        h = h_r[...]
        agg = hp0_r[...] + hp1_r[...]
        bn0 = b_r[0:1, :]
        bn1 = b_r[1:2, :]
        o = _silu(jnp.dot(h, Wn0a_r[...], preferred_element_type=jnp.float32)
                  + jnp.dot(agg, Wn0b_r[...],
                            preferred_element_type=jnp.float32) + bn0)
        o = jnp.dot(o, Wn1_r[...], preferred_element_type=jnp.float32) + bn1
        hn = h + o
        csum = cp0_r[...] + cp1_r[...]
        cnt = jnp.clip(csum[:, 3:4], 1.0, None)
        upd = csum / cnt
        lane = lax.broadcasted_iota(jnp.int32, (BN, CDIM), 1)
        co_new = co_r[...] + jnp.where(lane < 3, upd, 0.0)
        if last:
            hf_r, co_r_out = outs
            hf_r[...] = (jnp.dot(hn, Wax_r[...],
                                 preferred_element_type=jnp.float32)
                         + b_r[2:3, :])
            co_r_out[...] = co_new
        else:
            hn_r, co_r_out, ha_r, hb_r = outs
            hn_r[...] = hn
            co_r_out[...] = co_new
            ha_r[...] = _pack_bf16(jnp.dot(hn, Wax_r[...],
                                           preferred_element_type=jnp.float32))
            hb_r[...] = _pack_bf16(jnp.dot(hn, Wbx_r[...],
                                           preferred_element_type=jnp.float32))

    grid = (NP // BN,)
    nd = pl.BlockSpec((BN, D), lambda i: (i, 0))
    ndc = pl.BlockSpec((BN, CDIM), lambda i: (i, 0))
    if last:
        out_specs = [nd, ndc]
        out_shape = [jax.ShapeDtypeStruct((NP, D), jnp.float32),
                     jax.ShapeDtypeStruct((NP, CDIM), jnp.float32)]
    else:
        ndp = pl.BlockSpec((BN, DP), lambda i: (i, 0))
        out_specs = [nd, ndc, ndp, ndp]
        out_shape = [jax.ShapeDtypeStruct((NP, D), jnp.float32),
                     jax.ShapeDtypeStruct((NP, CDIM), jnp.float32),
                     jax.ShapeDtypeStruct((NP, DP), jnp.int32),
                     jax.ShapeDtypeStruct((NP, DP), jnp.int32)]
    return pl.pallas_call(
        body,
        grid=grid,
        in_specs=[nd, nd, nd, ndc, ndc, ndc,
                  _full((D, D)), _full((D, D)), _full((D, D)),
                  _full((D, D)), _full((D, D)), _full((8, D))],
        out_specs=out_specs,
        out_shape=out_shape,
    )(h, hp0, hp1, co, cp0, cp1, Wn0a, Wn0b, Wn1, Wax, Wbx, bias)


def _init_tc(xp, Wemb, Wa0, Wb0, bias):
    def body(x_r, Wemb_r, Wa_r, Wb_r, b_r, h_r, ha_r, hb_r):
        h = (jnp.dot(x_r[...], Wemb_r[...],
                     preferred_element_type=jnp.float32) + b_r[0:1, :])
        h_r[...] = h
        ha_r[...] = _pack_bf16(jnp.dot(h, Wa_r[...],
                                       preferred_element_type=jnp.float32))
        hb_r[...] = _pack_bf16(jnp.dot(h, Wb_r[...],
                                       preferred_element_type=jnp.float32))

    grid = (NP // BN,)
    nd = pl.BlockSpec((BN, D), lambda i: (i, 0))
    ndp = pl.BlockSpec((BN, DP), lambda i: (i, 0))
    return pl.pallas_call(
        body,
        grid=grid,
        in_specs=[nd, _full((D, D)), _full((D, D)), _full((D, D)),
                  _full((8, D))],
        out_specs=[nd, ndp, ndp],
        out_shape=[jax.ShapeDtypeStruct((NP, D), jnp.float32),
                   jax.ShapeDtypeStruct((NP, DP), jnp.int32),
                   jax.ShapeDtypeStruct((NP, DP), jnp.int32)],
    )(xp, Wemb, Wa0, Wb0, bias)


def _bias_stack(rows):
    stack = jnp.stack(rows, axis=0)
    return jnp.pad(stack, ((0, 8 - stack.shape[0]), (0, 0)))


def kernel(x, pos, edge_index, edge_attr, params):
    f32 = jnp.float32
    xp = jnp.pad(x.astype(f32), ((0, NP - N), (0, 0)))
    co = jnp.zeros((NP, CDIM), f32).at[:N, :3].set(pos.astype(f32))
    row = edge_index[0].astype(jnp.int32)
    col = edge_index[1].astype(jnp.int32)
    rowp = jnp.pad(row, (0, EP - E), constant_values=N)
    idxr3 = rowp.reshape(NW, KB, K)
    idxr2 = rowp.reshape(NW, EW // 64, 64)
    idxc3 = jnp.pad(col, (0, EP - E), constant_values=N).reshape(NW, KB, K)
    eap = jnp.pad(edge_attr.astype(f32), ((0, EP - E), (0, 0)))

    layers = params["layers"]
    l0 = layers[0]
    W0 = l0["edge_mlp0"]["W"]
    init_bias = _bias_stack([params["emb_in"]["b"]])
    h, ha, hb = _init_tc(xp, params["emb_in"]["W"],
                         W0[:D, :], W0[D:2 * D, :], init_bias)

    n_layers = len(layers)
    for li, lp in enumerate(layers):
        W0 = lp["edge_mlp0"]["W"]
        edge_bias = _bias_stack([
            lp["edge_mlp0"]["b"],
            lp["edge_mlp1"]["b"],
            lp["coord_mlp0"]["b"],
            lp["coord_mlp1"]["W"][:, 0],
            W0[2 * D, :],
        ])
        ghab, gcd = _sc_gather(ha, hb, co, idxr3, idxc3)
        m, wcd = _edge_tc(ghab, gcd, eap,
                          W0[2 * D + 1:, :], lp["edge_mlp1"]["W"],
                          lp["coord_mlp0"]["W"], edge_bias)
        hpart, cpart = _sc_scatter(m, wcd, idxr2)
        Wn = lp["node_mlp0"]["W"]
        last = li == n_layers - 1
        if last:
            node_bias = _bias_stack([lp["node_mlp0"]["b"],
                                     lp["node_mlp1"]["b"],
                                     params["emb_out"]["b"]])
            hf, co = _node_tc(h, hpart[0], hpart[1], co, cpart[0], cpart[1],
                              Wn[:D, :], Wn[D:, :], lp["node_mlp1"]["W"],
                              params["emb_out"]["W"], params["emb_out"]["W"],
                              node_bias, True)
        else:
            node_bias = _bias_stack([lp["node_mlp0"]["b"],
                                     lp["node_mlp1"]["b"]])
            Wnext = layers[li + 1]["edge_mlp0"]["W"]
            h, co, ha, hb = _node_tc(h, hpart[0], hpart[1], co,
                                     cpart[0], cpart[1],
                                     Wn[:D, :], Wn[D:, :],
                                     lp["node_mlp1"]["W"],
                                     Wnext[:D, :], Wnext[D:2 * D, :],
                                     node_bias, False)

    return co[:N, :3], hf[:N, :]
